# baseline (speedup 1.0000x reference)
import jax
import jax.numpy as jnp
from jax.experimental import pallas as pl
from jax.experimental.pallas import tpu as pltpu

B = 64
N = 1024
IN_FEAT = 192
HIDDEN = 128
CODE_DIM = 32
CODEBOOK_SIZE = 256
OUT_FEAT = 192
UNROLL = 8


def _body(xt_ref, wd_ref, bd_ref, wi_ref, bi_ref, cb_ref, wo_ref, bo_ref,
          wu_ref, bu_ref, out_ref, idx_ref, table_ref):
    def mm(a, b, dims):
        return jax.lax.dot_general(a.astype(jnp.bfloat16),
                                   b.astype(jnp.bfloat16), dims,
                                   preferred_element_type=jnp.float32)

    @pl.when(pl.program_id(0) == 0)
    def _():
        cb0 = cb_ref[...]
        t0 = mm(cb0, wo_ref[...], (((1,), (1,)), ((), ()))) + bo_ref[...]
        t1 = mm(t0, wu_ref[...], (((1,), (1,)), ((), ()))) + bu_ref[...]
        table_ref[...] = jnp.clip(t1, -1.0, 1.0)

    cb = cb_ref[...]
    c2 = jax.lax.dot_general(cb * cb, jnp.ones((CODE_DIM, 1), jnp.float32),
                             (((1,), (0,)), ((), ())),
                             precision=jax.lax.Precision.HIGHEST,
                             preferred_element_type=jnp.float32)
    table16 = table_ref[...].astype(jnp.bfloat16)
    for u in range(UNROLL):
        xt = xt_ref[u]
        h = mm(wd_ref[...], xt, (((1,), (0,)), ((), ()))) + bd_ref[...]
        z = mm(wi_ref[...], h, (((1,), (0,)), ((), ()))) + bi_ref[...]
        scores = mm(cb, z, (((1,), (0,)), ((), ())))
        zz = jnp.sum(z * z, axis=0, keepdims=True)
        dist = (zz - 2.0 * scores) + c2
        dmin = jnp.min(dist, axis=0, keepdims=True)
        row = jax.lax.broadcasted_iota(jnp.int32, dist.shape, 0)
        idx = jnp.min(jnp.where(dist == dmin, row, CODEBOOK_SIZE), axis=0,
                      keepdims=True)
        idx_ref[u] = idx
        onehot = (row == idx).astype(jnp.bfloat16)
        out_ref[u] = jax.lax.dot_general(table16, onehot,
                                         (((0,), (0,)), ((), ())),
                                         preferred_element_type=jnp.float32)


def kernel(x, W_down, b_down, W_in, b_in, codebook, W_out, b_out, W_up, b_up):
    xt = jnp.transpose(x, (0, 2, 1))
    full = lambda shape: pl.BlockSpec(shape, lambda i: (0,) * len(shape))
    out_t, idx = pl.pallas_call(
        _body,
        grid=(B // UNROLL,),
        in_specs=[
            pl.BlockSpec((UNROLL, IN_FEAT, N), lambda i: (i, 0, 0)),
            full((HIDDEN, IN_FEAT)),
            full((HIDDEN, 1)),
            full((CODE_DIM, HIDDEN)),
            full((CODE_DIM, 1)),
            full((CODEBOOK_SIZE, CODE_DIM)),
            full((HIDDEN, CODE_DIM)),
            full((1, HIDDEN)),
            full((OUT_FEAT, HIDDEN)),
            full((1, OUT_FEAT)),
        ],
        out_specs=[
            pl.BlockSpec((UNROLL, OUT_FEAT, N), lambda i: (i, 0, 0)),
            pl.BlockSpec((UNROLL, 1, N), lambda i: (i, 0, 0)),
        ],
        out_shape=[
            jax.ShapeDtypeStruct((B, OUT_FEAT, N), jnp.float32),
            jax.ShapeDtypeStruct((B, 1, N), jnp.int32),
        ],
        scratch_shapes=[pltpu.VMEM((CODEBOOK_SIZE, OUT_FEAT), jnp.float32)],
    )(xt, W_down, b_down.reshape(HIDDEN, 1), W_in, b_in.reshape(CODE_DIM, 1),
      codebook, W_out, b_out.reshape(1, HIDDEN), W_up, b_up.reshape(1, OUT_FEAT))
    out = jnp.transpose(out_t, (0, 2, 1))
    indices = idx.reshape(B, N)
    commit_loss = jnp.zeros((), dtype=jnp.float32)
    return (out, indices, commit_loss)


# speedup vs baseline: 1.0040x; 1.0040x over previous
import jax
import jax.numpy as jnp
from jax.experimental import pallas as pl
from jax.experimental.pallas import tpu as pltpu

B = 64
N = 1024
IN_FEAT = 192
HIDDEN = 128
CODE_DIM = 32
CODEBOOK_SIZE = 256
OUT_FEAT = 192
UNROLL = 4


def _body(xt_ref, wd_ref, bd_ref, wi_ref, bi_ref, cb_ref, wo_ref, bo_ref,
          wu_ref, bu_ref, out_ref, idx_ref, table_ref):
    def mm(a, b, dims):
        return jax.lax.dot_general(a.astype(jnp.bfloat16),
                                   b.astype(jnp.bfloat16), dims,
                                   preferred_element_type=jnp.float32)

    @pl.when(pl.program_id(0) == 0)
    def _():
        cb0 = cb_ref[...]
        t0 = mm(cb0, wo_ref[...], (((1,), (1,)), ((), ()))) + bo_ref[...]
        t1 = mm(t0, wu_ref[...], (((1,), (1,)), ((), ()))) + bu_ref[...]
        table_ref[...] = jnp.clip(t1, -1.0, 1.0)

    cb = cb_ref[...]
    c2 = jax.lax.dot_general(cb * cb, jnp.ones((CODE_DIM, 1), jnp.float32),
                             (((1,), (0,)), ((), ())),
                             precision=jax.lax.Precision.HIGHEST,
                             preferred_element_type=jnp.float32)
    table16 = table_ref[...].astype(jnp.bfloat16)
    for u in range(UNROLL):
        xt = xt_ref[u]
        h = mm(wd_ref[...], xt, (((1,), (0,)), ((), ()))) + bd_ref[...]
        z = mm(wi_ref[...], h, (((1,), (0,)), ((), ()))) + bi_ref[...]
        scores = mm(cb, z, (((1,), (0,)), ((), ())))
        zz = jnp.sum(z * z, axis=0, keepdims=True)
        dist = (zz - 2.0 * scores) + c2
        dmin = jnp.min(dist, axis=0, keepdims=True)
        row = jax.lax.broadcasted_iota(jnp.int32, dist.shape, 0)
        idx = jnp.min(jnp.where(dist == dmin, row, CODEBOOK_SIZE), axis=0,
                      keepdims=True)
        idx_ref[u] = idx
        onehot = (row == idx).astype(jnp.bfloat16)
        out_ref[u] = jax.lax.dot_general(table16, onehot,
                                         (((0,), (0,)), ((), ())),
                                         preferred_element_type=jnp.float32)


def kernel(x, W_down, b_down, W_in, b_in, codebook, W_out, b_out, W_up, b_up):
    xt = jnp.transpose(x, (0, 2, 1))
    full = lambda shape: pl.BlockSpec(shape, lambda i: (0,) * len(shape))
    out_t, idx = pl.pallas_call(
        _body,
        grid=(B // UNROLL,),
        in_specs=[
            pl.BlockSpec((UNROLL, IN_FEAT, N), lambda i: (i, 0, 0)),
            full((HIDDEN, IN_FEAT)),
            full((HIDDEN, 1)),
            full((CODE_DIM, HIDDEN)),
            full((CODE_DIM, 1)),
            full((CODEBOOK_SIZE, CODE_DIM)),
            full((HIDDEN, CODE_DIM)),
            full((1, HIDDEN)),
            full((OUT_FEAT, HIDDEN)),
            full((1, OUT_FEAT)),
        ],
        out_specs=[
            pl.BlockSpec((UNROLL, OUT_FEAT, N), lambda i: (i, 0, 0)),
            pl.BlockSpec((UNROLL, 1, N), lambda i: (i, 0, 0)),
        ],
        out_shape=[
            jax.ShapeDtypeStruct((B, OUT_FEAT, N), jnp.float32),
            jax.ShapeDtypeStruct((B, 1, N), jnp.int32),
        ],
        scratch_shapes=[pltpu.VMEM((CODEBOOK_SIZE, OUT_FEAT), jnp.float32)],
    )(xt, W_down, b_down.reshape(HIDDEN, 1), W_in, b_in.reshape(CODE_DIM, 1),
      codebook, W_out, b_out.reshape(1, HIDDEN), W_up, b_up.reshape(1, OUT_FEAT))
    out = jnp.transpose(out_t, (0, 2, 1))
    indices = idx.reshape(B, N)
    commit_loss = jnp.zeros((), dtype=jnp.float32)
    return (out, indices, commit_loss)
